# baseline (speedup 1.0000x reference)
import functools

import jax
import jax.numpy as jnp
from jax import lax
from jax.experimental import pallas as pl
from jax.experimental.pallas import tpu as pltpu
from jax.experimental.pallas import tpu_sc as plsc

VOCAB = 1000
EMB = 128
BATCH = 4096
HIST = 50

NC = 2
NS = 16
NW = NC * NS
NSPLIT = 4
CB = BATCH // NSPLIT
BPW = CB // NW
NB = 4
NP = BPW // NB


def _embed_chunk(idx, W):
    mesh = plsc.VectorSubcoreMesh(core_axis_name="c", subcore_axis_name="s")

    @functools.partial(
        pl.kernel,
        mesh=mesh,
        out_type=jax.ShapeDtypeStruct((CB, HIST, EMB), jnp.float32),
        scratch_types=[
            pltpu.VMEM((BPW, HIST), jnp.int32),
            pltpu.VMEM((NB, HIST, EMB), jnp.float32),
            pltpu.VMEM_SHARED((VOCAB, EMB), jnp.float32),
            pltpu.SemaphoreType.DMA((NB,)),
            pltpu.SemaphoreType.DMA((NB,)),
        ],
    )
    def k(table_hbm, idx_hbm, out_hbm, idx_v, bufs, tab_sh, gsem, ssem):
        cid = lax.axis_index("c")
        sid = lax.axis_index("s")
        wid = sid * NC + cid
        base = wid * BPW

        @pl.when(sid == 0)
        def _():
            pltpu.sync_copy(table_hbm, tab_sh)

        pltpu.sync_copy(idx_hbm.at[pl.ds(base, BPW)], idx_v)
        plsc.subcore_barrier()

        def gather(j, b):
            pltpu.async_copy(tab_sh.at[idx_v.at[j]], bufs.at[b], gsem.at[b])

        def store(j, b):
            dst = out_hbm.at[base + j]
            pltpu.async_copy(bufs.at[b], dst, ssem.at[b])
            return dst

        for b in range(NB):
            gather(b, b)

        def body(p, carry):
            for b in range(NB):
                j = p * NB + b
                pltpu.make_async_copy(
                    tab_sh.at[idx_v.at[j]], bufs.at[b], gsem.at[b]
                ).wait()
                dst = store(j, b)
                pltpu.make_async_copy(bufs.at[b], dst, ssem.at[b]).wait()
                gather(j + NB, b)
            return carry

        lax.fori_loop(0, NP - 1, body, 0)

        for b in range(NB):
            j = (NP - 1) * NB + b
            pltpu.make_async_copy(
                tab_sh.at[idx_v.at[j]], bufs.at[b], gsem.at[b]
            ).wait()
            dst = store(j, b)
            pltpu.make_async_copy(bufs.at[b], dst, ssem.at[b]).wait()

    return k(W, idx)


def kernel(input, W):
    parts = [
        _embed_chunk(input[i * CB:(i + 1) * CB], W) for i in range(NSPLIT)
    ]
    return jnp.concatenate(parts, axis=0)


# speedup vs baseline: 1.1408x; 1.1408x over previous
import functools

import jax
import jax.numpy as jnp
from jax import lax
from jax.experimental import pallas as pl
from jax.experimental.pallas import tpu as pltpu
from jax.experimental.pallas import tpu_sc as plsc

VOCAB = 1000
EMB = 128
BATCH = 4096
HIST = 50

NC = 2
NS = 16
NW = NC * NS
NSPLIT = 2
CB = BATCH // NSPLIT
BPW = CB // NW
NB = 4
NP = BPW // NB


def _embed_chunk(idx, W):
    mesh = plsc.VectorSubcoreMesh(core_axis_name="c", subcore_axis_name="s")

    @functools.partial(
        pl.kernel,
        mesh=mesh,
        out_type=jax.ShapeDtypeStruct((CB, HIST, EMB), jnp.float32),
        scratch_types=[
            pltpu.VMEM((BPW, HIST), jnp.int32),
            pltpu.VMEM((NB, HIST, EMB), jnp.float32),
            pltpu.VMEM_SHARED((VOCAB, EMB), jnp.float32),
            pltpu.SemaphoreType.DMA((NB,)),
            pltpu.SemaphoreType.DMA((NB,)),
        ],
    )
    def k(table_hbm, idx_hbm, out_hbm, idx_v, bufs, tab_sh, gsem, ssem):
        cid = lax.axis_index("c")
        sid = lax.axis_index("s")
        wid = sid * NC + cid
        base = wid * BPW

        @pl.when(sid == 0)
        def _():
            pltpu.sync_copy(table_hbm, tab_sh)

        pltpu.sync_copy(idx_hbm.at[pl.ds(base, BPW)], idx_v)
        plsc.subcore_barrier()

        def gather(j, b):
            pltpu.async_copy(tab_sh.at[idx_v.at[j]], bufs.at[b], gsem.at[b])

        def store(j, b):
            dst = out_hbm.at[base + j]
            pltpu.async_copy(bufs.at[b], dst, ssem.at[b])
            return dst

        for b in range(NB):
            gather(b, b)

        def body(p, carry):
            for b in range(NB):
                j = p * NB + b
                pltpu.make_async_copy(
                    tab_sh.at[idx_v.at[j]], bufs.at[b], gsem.at[b]
                ).wait()
                dst = store(j, b)
                pltpu.make_async_copy(bufs.at[b], dst, ssem.at[b]).wait()
                gather(j + NB, b)
            return carry

        lax.fori_loop(0, NP - 1, body, 0)

        for b in range(NB):
            j = (NP - 1) * NB + b
            pltpu.make_async_copy(
                tab_sh.at[idx_v.at[j]], bufs.at[b], gsem.at[b]
            ).wait()
            dst = store(j, b)
            pltpu.make_async_copy(bufs.at[b], dst, ssem.at[b]).wait()

    return k(W, idx)


def kernel(input, W):
    parts = [
        _embed_chunk(input[i * CB:(i + 1) * CB], W) for i in range(NSPLIT)
    ]
    return jnp.concatenate(parts, axis=0)


# speedup vs baseline: 2.0366x; 1.7853x over previous
import functools

import jax
import jax.numpy as jnp
from jax import lax
from jax.experimental import pallas as pl
from jax.experimental.pallas import tpu as pltpu
from jax.experimental.pallas import tpu_sc as plsc

VOCAB = 1000
EMB = 128
BATCH = 4096
HIST = 50

NC = 2
NS = 16
NW = NC * NS
BPW = BATCH // NW
PAIR = 2
CHN = BPW // PAIR
NB = 4
NP = CHN // NB


def _embed(idx, W):
    mesh = plsc.VectorSubcoreMesh(core_axis_name="c", subcore_axis_name="s")

    @functools.partial(
        pl.kernel,
        mesh=mesh,
        out_type=jax.ShapeDtypeStruct((BATCH, HIST, EMB), jnp.float32),
        scratch_types=[
            pltpu.VMEM((BPW, HIST), jnp.int32),
            pltpu.VMEM((NB, PAIR, HIST, EMB), jnp.float32),
            pltpu.VMEM_SHARED((VOCAB, EMB), jnp.float32),
            pltpu.SemaphoreType.DMA((NB,)),
            pltpu.SemaphoreType.DMA((NB,)),
        ],
    )
    def k(table_hbm, idx_hbm, out_hbm, idx_v, bufs, tab_sh, gsem, ssem):
        cid = lax.axis_index("c")
        sid = lax.axis_index("s")
        wid = sid * NC + cid
        base = wid * BPW

        @pl.when(sid == 0)
        def _():
            pltpu.sync_copy(table_hbm, tab_sh)

        pltpu.sync_copy(idx_hbm.at[pl.ds(base, BPW)], idx_v)
        plsc.subcore_barrier()

        def gather(j, b):
            for t in range(PAIR):
                pltpu.async_copy(
                    tab_sh.at[idx_v.at[j * PAIR + t]],
                    bufs.at[b].at[t],
                    gsem.at[b],
                )

        def gwait(j, b):
            for t in range(PAIR):
                pltpu.make_async_copy(
                    tab_sh.at[idx_v.at[j * PAIR + t]],
                    bufs.at[b].at[t],
                    gsem.at[b],
                ).wait()

        def store(j, b):
            dst = out_hbm.at[pl.ds(base + j * PAIR, PAIR)]
            pltpu.async_copy(bufs.at[b], dst, ssem.at[b])
            return dst

        for b in range(NB):
            gather(b, b)

        def body(p, carry):
            for b in range(NB):
                j = p * NB + b
                gwait(j, b)
                dst = store(j, b)
                pltpu.make_async_copy(bufs.at[b], dst, ssem.at[b]).wait()
                gather(j + NB, b)
            return carry

        lax.fori_loop(0, NP - 1, body, 0)

        for b in range(NB):
            j = (NP - 1) * NB + b
            gwait(j, b)
            dst = store(j, b)
            pltpu.make_async_copy(bufs.at[b], dst, ssem.at[b]).wait()

    return k(W, idx)


def kernel(input, W):
    return _embed(input, W)


# speedup vs baseline: 4.3686x; 2.1451x over previous
import functools

import jax
import jax.numpy as jnp
from jax import lax
from jax.experimental import pallas as pl
from jax.experimental.pallas import tpu as pltpu
from jax.experimental.pallas import tpu_sc as plsc

VOCAB = 1000
EMB = 128
BATCH = 4096
HIST = 50

B = BATCH * HIST
NC = 2
NS = 16
NW = NC * NS
BPW = B // NW
CH = 128
NCH = BPW // CH
NB = 4
NP = NCH // NB
TAIL = NCH - NP * NB


def _embed_flat(idx3, W):
    mesh = plsc.VectorSubcoreMesh(core_axis_name="c", subcore_axis_name="s")

    @functools.partial(
        pl.kernel,
        mesh=mesh,
        out_type=jax.ShapeDtypeStruct((B, EMB), jnp.float32),
        scratch_types=[
            pltpu.VMEM((NCH, CH), jnp.int32),
            pltpu.VMEM((NB, CH, EMB), jnp.float32),
            pltpu.VMEM_SHARED((VOCAB, EMB), jnp.float32),
            pltpu.SemaphoreType.DMA((NB,)),
            pltpu.SemaphoreType.DMA((NB,)),
        ],
    )
    def k(table_hbm, idx_hbm, out_hbm, idx_v, bufs, tab_sh, gsem, ssem):
        cid = lax.axis_index("c")
        sid = lax.axis_index("s")
        wid = sid * NC + cid
        base = wid * BPW

        @pl.when(sid == 0)
        def _():
            pltpu.sync_copy(table_hbm, tab_sh)

        pltpu.sync_copy(idx_hbm.at[wid], idx_v)
        plsc.subcore_barrier()

        def gather(j, b):
            pltpu.async_copy(tab_sh.at[idx_v.at[j]], bufs.at[b], gsem.at[b])

        def store(j, b):
            dst = out_hbm.at[pl.ds(base + j * CH, CH)]
            pltpu.async_copy(bufs.at[b], dst, ssem.at[b])
            return dst

        for b in range(NB):
            gather(b, b)

        def body(p, carry):
            for b in range(NB):
                j = p * NB + b
                pltpu.make_async_copy(
                    tab_sh.at[idx_v.at[j]], bufs.at[b], gsem.at[b]
                ).wait()
                dst = store(j, b)
                pltpu.make_async_copy(bufs.at[b], dst, ssem.at[b]).wait()
                gather(j + NB, b)
            return carry

        lax.fori_loop(0, NP - 1, body, 0)

        for t in range(NB + TAIL):
            j = (NP - 1) * NB + t
            b = t % NB
            pltpu.make_async_copy(
                tab_sh.at[idx_v.at[j]], bufs.at[b], gsem.at[b]
            ).wait()
            dst = store(j, b)
            pltpu.make_async_copy(bufs.at[b], dst, ssem.at[b]).wait()
            if t + NB < NB + TAIL:
                gather(j + NB, b)

    return k(W, idx3)


def kernel(input, W):
    idx3 = input.T.reshape(NW, NCH, CH)
    out = _embed_flat(idx3, W)
    return out.reshape(HIST, BATCH, EMB).transpose(1, 0, 2)
